# Initial kernel scaffold; baseline (speedup 1.0000x reference)
import jax
import jax.numpy as jnp
from jax.experimental import pallas as pl


def kernel(input_ids, emb_W, gru_Wih, gru_Whh, gru_bih, gru_bhh, q_W, q_b, k_W, k_b, gate_W, gate_b, hfc_W, hfc_b, hp_W, hp_b, out_bias, mem_scale):
    raise NotImplementedError("write your pallas kernel here")


# speedup vs baseline: 11.7028x; 11.7028x over previous
import functools

import jax
import jax.numpy as jnp
from jax.experimental import pallas as pl
from jax.experimental.pallas import tpu as pltpu

F32 = jnp.float32
BF16 = jnp.bfloat16


def _gru_kernel(x_ref, wih_ref, whh_ref, bih_ref, bhh_ref, out_ref, h_ref,
                *, ct, b, h):
    i = pl.program_id(0)

    @pl.when(i == 0)
    def _():
        h_ref[...] = jnp.zeros_like(h_ref)

    gx = jnp.dot(x_ref[0], wih_ref[...], preferred_element_type=F32)
    gx = gx + bih_ref[...]
    for t in range(ct):
        gxt = gx[t * b:(t + 1) * b, :]
        hprev = h_ref[...]
        gh = jnp.dot(hprev, whh_ref[...], preferred_element_type=F32)
        gh = gh + bhh_ref[...]
        r = jax.nn.sigmoid(gxt[:, :h] + gh[:, :h])
        z = jax.nn.sigmoid(gxt[:, h:2 * h] + gh[:, h:2 * h])
        n = jnp.tanh(gxt[:, 2 * h:] + r * gh[:, 2 * h:])
        hnew = (1.0 - z) * n + z * hprev
        h_ref[...] = hnew
        out_ref[:, t, :] = hnew


def _proj_kernel(s_ref, hfc_ref, hfcb_ref, hp_ref, hpb_ref,
                 qw_ref, qb_ref, kw_ref, kb_ref,
                 gw_ref, gb_ref, ms_ref,
                 base_ref, q_ref, k_ref, g_ref):
    s = s_ref[...]
    sb = s.astype(BF16)
    hf = jnp.dot(sb, hfc_ref[...], preferred_element_type=F32) + hfcb_ref[...]
    hf = jnp.square(jnp.maximum(hf, 0.0))
    base = jnp.dot(hf.astype(BF16), hp_ref[...],
                   preferred_element_type=F32) + hpb_ref[...]
    base_ref[...] = base.astype(BF16)
    q = jnp.dot(sb, qw_ref[...], preferred_element_type=F32) + qb_ref[...]
    q_ref[...] = q.astype(BF16)
    k = jnp.dot(sb, kw_ref[...], preferred_element_type=F32) + kb_ref[...]
    k_ref[...] = k.astype(BF16)
    gpre = jnp.sum(s * gw_ref[...], axis=-1, keepdims=True) + gb_ref[...]
    g_ref[...] = jax.nn.sigmoid(gpre) * ms_ref[...]


def _attn_kernel(q_ref, kt_ref, g_ref, ga_ref, *, tq, t, scale):
    qt = pl.program_id(1)
    s = jnp.dot(q_ref[0], kt_ref[0], preferred_element_type=F32) * scale
    qi = qt * tq + jax.lax.broadcasted_iota(jnp.int32, (tq, t), 0)
    ki = jax.lax.broadcasted_iota(jnp.int32, (tq, t), 1)
    mask = ki < qi
    m = jnp.max(jnp.where(mask, s, -3e38), axis=-1, keepdims=True)
    p = jnp.where(mask, jnp.exp(s - m), 0.0)
    denom = jnp.maximum(jnp.sum(p, axis=-1, keepdims=True), 1e-6)
    ga = (p / denom) * g_ref[0]
    ga_ref[0] = ga.astype(BF16)


def _decode_kernel(base_ref, embt_ref, ga_ref, ids_ref, bias_ref, out_ref,
                   *, t, vt):
    v = pl.program_id(1)
    c = jnp.dot(base_ref[0], embt_ref[...], preferred_element_type=F32)
    col = jax.lax.broadcasted_iota(jnp.int32, (t, vt), 1) + v * vt
    onehot = jnp.where(ids_ref[0] == col, 1.0, 0.0).astype(BF16)
    cb = jnp.dot(ga_ref[0], onehot, preferred_element_type=F32)
    out_ref[0] = c + bias_ref[...] + cb


def kernel(input_ids, emb_W, gru_Wih, gru_Whh, gru_bih, gru_bhh,
           q_W, q_b, k_W, k_b, gate_W, gate_b,
           hfc_W, hfc_b, hp_W, hp_b, out_bias, mem_scale):
    B, T = input_ids.shape
    V, E = emb_W.shape
    H = gru_Whh.shape[1]
    M = q_W.shape[0]
    CT = 8
    TQ = 256
    VT = 1280
    TR = 512

    x = emb_W[input_ids]
    x_c = jnp.transpose(x, (1, 0, 2)).reshape(T // CT, CT * B, E)
    wih_t = gru_Wih.T
    whh_t = gru_Whh.T

    states = pl.pallas_call(
        functools.partial(_gru_kernel, ct=CT, b=B, h=H),
        grid=(T // CT,),
        in_specs=[
            pl.BlockSpec((1, CT * B, E), lambda i: (i, 0, 0)),
            pl.BlockSpec((E, 3 * H), lambda i: (0, 0)),
            pl.BlockSpec((H, 3 * H), lambda i: (0, 0)),
            pl.BlockSpec((1, 3 * H), lambda i: (0, 0)),
            pl.BlockSpec((1, 3 * H), lambda i: (0, 0)),
        ],
        out_specs=pl.BlockSpec((B, CT, H), lambda i: (0, i, 0)),
        out_shape=jax.ShapeDtypeStruct((B, T, H), F32),
        scratch_shapes=[pltpu.VMEM((B, H), F32)],
        compiler_params=pltpu.CompilerParams(
            dimension_semantics=("arbitrary",),
            vmem_limit_bytes=50 * 1024 * 1024,
        ),
    )(x_c, wih_t, whh_t, gru_bih.reshape(1, 3 * H), gru_bhh.reshape(1, 3 * H))

    s2 = states.reshape(B * T, H)
    base, q, k, g = pl.pallas_call(
        _proj_kernel,
        grid=(B * T // TR,),
        in_specs=[
            pl.BlockSpec((TR, H), lambda i: (i, 0)),
            pl.BlockSpec((H, 4 * E), lambda i: (0, 0)),
            pl.BlockSpec((1, 4 * E), lambda i: (0, 0)),
            pl.BlockSpec((4 * E, E), lambda i: (0, 0)),
            pl.BlockSpec((1, E), lambda i: (0, 0)),
            pl.BlockSpec((H, M), lambda i: (0, 0)),
            pl.BlockSpec((1, M), lambda i: (0, 0)),
            pl.BlockSpec((H, M), lambda i: (0, 0)),
            pl.BlockSpec((1, M), lambda i: (0, 0)),
            pl.BlockSpec((1, H), lambda i: (0, 0)),
            pl.BlockSpec((1, 1), lambda i: (0, 0)),
            pl.BlockSpec((1, 1), lambda i: (0, 0)),
        ],
        out_specs=[
            pl.BlockSpec((TR, E), lambda i: (i, 0)),
            pl.BlockSpec((TR, M), lambda i: (i, 0)),
            pl.BlockSpec((TR, M), lambda i: (i, 0)),
            pl.BlockSpec((TR, 1), lambda i: (i, 0)),
        ],
        out_shape=[
            jax.ShapeDtypeStruct((B * T, E), BF16),
            jax.ShapeDtypeStruct((B * T, M), BF16),
            jax.ShapeDtypeStruct((B * T, M), BF16),
            jax.ShapeDtypeStruct((B * T, 1), F32),
        ],
        compiler_params=pltpu.CompilerParams(
            dimension_semantics=("parallel",),
            vmem_limit_bytes=50 * 1024 * 1024,
        ),
    )(s2, hfc_W.T.astype(BF16), hfc_b.reshape(1, 4 * E),
      hp_W.T.astype(BF16), hp_b.reshape(1, E),
      q_W.T.astype(BF16), q_b.reshape(1, M),
      k_W.T.astype(BF16), k_b.reshape(1, M),
      gate_W.reshape(1, H), gate_b.reshape(1, 1),
      mem_scale.reshape(1, 1))

    q3 = q.reshape(B, T, M)
    kt3 = jnp.transpose(k.reshape(B, T, M), (0, 2, 1))
    g3 = g.reshape(B, T, 1)
    ga = pl.pallas_call(
        functools.partial(_attn_kernel, tq=TQ, t=T, scale=1.0 / (M ** 0.5)),
        grid=(B, T // TQ),
        in_specs=[
            pl.BlockSpec((1, TQ, M), lambda b, i: (b, i, 0)),
            pl.BlockSpec((1, M, T), lambda b, i: (b, 0, 0)),
            pl.BlockSpec((1, TQ, 1), lambda b, i: (b, i, 0)),
        ],
        out_specs=pl.BlockSpec((1, TQ, T), lambda b, i: (b, i, 0)),
        out_shape=jax.ShapeDtypeStruct((B, T, T), BF16),
        compiler_params=pltpu.CompilerParams(
            dimension_semantics=("parallel", "parallel"),
            vmem_limit_bytes=50 * 1024 * 1024,
        ),
    )(q3, kt3, g3)

    base3 = base.reshape(B, T, E)
    embt = emb_W.T.astype(BF16)
    ids_c = input_ids.reshape(B, T, 1)
    logits = pl.pallas_call(
        functools.partial(_decode_kernel, t=T, vt=VT),
        grid=(B, V // VT),
        in_specs=[
            pl.BlockSpec((1, T, E), lambda b, v: (b, 0, 0)),
            pl.BlockSpec((E, VT), lambda b, v: (0, v)),
            pl.BlockSpec((1, T, T), lambda b, v: (b, 0, 0)),
            pl.BlockSpec((1, T, 1), lambda b, v: (b, 0, 0)),
            pl.BlockSpec((1, VT), lambda b, v: (0, v)),
        ],
        out_specs=pl.BlockSpec((1, T, VT), lambda b, v: (b, 0, v)),
        out_shape=jax.ShapeDtypeStruct((B, T, V), F32),
        compiler_params=pltpu.CompilerParams(
            dimension_semantics=("parallel", "arbitrary"),
            vmem_limit_bytes=50 * 1024 * 1024,
        ),
    )(base3, embt, ga, ids_c, out_bias.reshape(1, V))
    return logits
